# Initial kernel scaffold; baseline (speedup 1.0000x reference)
import jax
import jax.numpy as jnp
from jax.experimental import pallas as pl


def kernel(query_features, class_embeddings):
    raise NotImplementedError("write your pallas kernel here")


# speedup vs baseline: 10.4740x; 10.4740x over previous
import functools

import jax
import jax.numpy as jnp
from jax.experimental import pallas as pl

_NUM_CLASSES = 200
_QPC = 40
_EMBED_DIM = 256
_QB = 320
_G = _QB // _QPC


def _logits_block(q_ref, ce_ref, out_ref):
    j = pl.program_id(1)
    q = q_ref[0]
    qq = jnp.sum(q * q, axis=1, keepdims=True)
    rq = jax.lax.rsqrt(jnp.maximum(qq, 1e-24))
    ce_sub = ce_ref[pl.ds(j * _G, _G), :]
    ee = jnp.sum(ce_sub * ce_sub, axis=1)
    rce = jax.lax.rsqrt(jnp.maximum(ee, 1e-24))
    bgv = ce_ref[_NUM_CLASSES:_NUM_CLASSES + 1, :]
    bb = jnp.sum(bgv * bgv)
    rbg = jax.lax.rsqrt(jnp.maximum(bb, 1e-24))
    s = jax.lax.dot_general(q, ce_sub, (((1,), (1,)), ((), ())),
                            preferred_element_type=jnp.float32)
    s = s * rq * rce[None, :]
    row = jax.lax.broadcasted_iota(jnp.int32, (_QB, 1), 0)
    localg = jax.lax.broadcasted_iota(jnp.int32, (_QB, _G), 1)
    class_sim = jnp.sum(jnp.where(localg == row // _QPC, s, 0.0),
                        axis=1, keepdims=True)
    bg_sim = jnp.sum(q * bgv, axis=1, keepdims=True) * rq * rbg
    col = jax.lax.broadcasted_iota(jnp.int32, (_QB, _NUM_CLASSES + 1), 1)
    rowcls = j * _G + row // _QPC
    out = jnp.where(col == rowcls, class_sim,
                    jnp.where(col == _NUM_CLASSES, bg_sim, -jnp.inf))
    out_ref[0] = out


@jax.jit
def kernel(query_features, class_embeddings):
    B, Q, D = query_features.shape
    grid = (B, Q // _QB)
    return pl.pallas_call(
        _logits_block,
        grid=grid,
        in_specs=[
            pl.BlockSpec((1, _QB, D), lambda b, j: (b, j, 0)),
            pl.BlockSpec((_NUM_CLASSES + 1, D), lambda b, j: (0, 0)),
        ],
        out_specs=pl.BlockSpec((1, _QB, _NUM_CLASSES + 1), lambda b, j: (b, j, 0)),
        out_shape=jax.ShapeDtypeStruct((B, Q, _NUM_CLASSES + 1), jnp.float32),
    )(query_features, class_embeddings)


# speedup vs baseline: 17.6875x; 1.6887x over previous
import functools

import jax
import jax.numpy as jnp
from jax.experimental import pallas as pl

_NUM_CLASSES = 200
_QPC = 40
_EMBED_DIM = 256
_QB = 1600
_G = _QB // _QPC


def _logits_block(q_ref, ce_ref, out_ref):
    j = pl.program_id(1)
    q = q_ref[0]
    qq = jnp.sum(q * q, axis=1, keepdims=True)
    rq = jax.lax.rsqrt(jnp.maximum(qq, 1e-24))
    ce_sub = ce_ref[pl.ds(j * _G, _G), :]
    ee = jnp.sum(ce_sub * ce_sub, axis=1)
    rce = jax.lax.rsqrt(jnp.maximum(ee, 1e-24))
    bgv = ce_ref[_NUM_CLASSES:_NUM_CLASSES + 1, :]
    bb = jnp.sum(bgv * bgv)
    rbg = jax.lax.rsqrt(jnp.maximum(bb, 1e-24))
    s = jax.lax.dot_general(q, ce_sub, (((1,), (1,)), ((), ())),
                            preferred_element_type=jnp.float32)
    s = s * rq * rce[None, :]
    row = jax.lax.broadcasted_iota(jnp.int32, (_QB, 1), 0)
    localg = jax.lax.broadcasted_iota(jnp.int32, (_QB, _G), 1)
    class_sim = jnp.sum(jnp.where(localg == row // _QPC, s, 0.0),
                        axis=1, keepdims=True)
    bg_sim = jnp.sum(q * bgv, axis=1, keepdims=True) * rq * rbg
    col = jax.lax.broadcasted_iota(jnp.int32, (_QB, _NUM_CLASSES + 1), 1)
    rowcls = j * _G + row // _QPC
    out = jnp.where(col == rowcls, class_sim,
                    jnp.where(col == _NUM_CLASSES, bg_sim, -jnp.inf))
    out_ref[0] = out


@jax.jit
def kernel(query_features, class_embeddings):
    B, Q, D = query_features.shape
    grid = (B, Q // _QB)
    return pl.pallas_call(
        _logits_block,
        grid=grid,
        in_specs=[
            pl.BlockSpec((1, _QB, D), lambda b, j: (b, j, 0)),
            pl.BlockSpec((_NUM_CLASSES + 1, D), lambda b, j: (0, 0)),
        ],
        out_specs=pl.BlockSpec((1, _QB, _NUM_CLASSES + 1), lambda b, j: (b, j, 0)),
        out_shape=jax.ShapeDtypeStruct((B, Q, _NUM_CLASSES + 1), jnp.float32),
    )(query_features, class_embeddings)
